# Initial kernel scaffold; baseline (speedup 1.0000x reference)
import jax
import jax.numpy as jnp
from jax.experimental import pallas as pl


def kernel(x, params):
    raise NotImplementedError("write your pallas kernel here")


# speedup vs baseline: 4.6532x; 4.6532x over previous
import functools

import jax
import jax.numpy as jnp
from jax.experimental import pallas as pl
from jax.experimental.pallas import tpu as pltpu

NUM_EXPERTS = 8
SIZE = 224
PATCH = 16
DIM = 384
DEPTH = 6
HEADS = 8
DIM_HEAD = 64
MLP_DIM = 512
NUM_CLASSES = 10
BATCH = 32
NPATCH = (SIZE // PATCH) ** 2
PATCH_DIM = 3 * PATCH * PATCH
INNER = HEADS * DIM_HEAD
SEQ = NPATCH + 1

PREC = jax.lax.Precision.HIGHEST

GATE_CHUNKS = 8
GATE_K = 3 * SIZE * SIZE // GATE_CHUNKS


def _mm(a, b):
    return jax.lax.dot_general(
        a, b, (((a.ndim - 1,), (0,)), ((), ())),
        precision=PREC, preferred_element_type=jnp.float32)


def _ln(x, g, b):
    m = jnp.mean(x, axis=-1, keepdims=True)
    v = jnp.mean((x - m) ** 2, axis=-1, keepdims=True)
    return (x - m) * jax.lax.rsqrt(v + 1e-5) * g + b


def _gate_kernel(xf_ref, gwt_ref, gb_ref, top1_ref, acc_ref):
    k = pl.program_id(0)

    @pl.when(k == 0)
    def _():
        acc_ref[...] = jnp.broadcast_to(gb_ref[...], (BATCH, NUM_EXPERTS))

    acc_ref[...] += jax.lax.dot_general(
        xf_ref[...], gwt_ref[...], (((1,), (1,)), ((), ())),
        precision=PREC, preferred_element_type=jnp.float32)

    @pl.when(k == GATE_CHUNKS - 1)
    def _():
        logits = acc_ref[...]
        m = jnp.max(logits, axis=1, keepdims=True)
        col = jax.lax.broadcasted_iota(jnp.int32, (BATCH, NUM_EXPERTS), 1)
        top1_ref[...] = jnp.min(
            jnp.where(logits == m, col, NUM_EXPERTS), axis=1, keepdims=True)


def _gate(xf, gwt, gb):
    return pl.pallas_call(
        _gate_kernel,
        grid=(GATE_CHUNKS,),
        in_specs=[
            pl.BlockSpec((BATCH, GATE_K), lambda k: (0, k)),
            pl.BlockSpec((NUM_EXPERTS, GATE_K), lambda k: (0, k)),
            pl.BlockSpec((1, NUM_EXPERTS), lambda k: (0, 0)),
        ],
        out_specs=pl.BlockSpec((BATCH, 1), lambda k: (0, 0)),
        out_shape=jax.ShapeDtypeStruct((BATCH, 1), jnp.int32),
        scratch_shapes=[pltpu.VMEM((BATCH, NUM_EXPERTS), jnp.float32)],
    )(xf, gwt, gb)


def _embed_kernel(se_ref, si_ref, xp_ref, g1_ref, b1_ref, pw_ref, pb_ref,
                  g2_ref, b2_ref, cls_ref, pos_ref, t0_ref):
    p = _ln(xp_ref[0], g1_ref[0, 0], b1_ref[0, 0])
    t = _mm(p, pw_ref[0]) + pb_ref[0, 0]
    t = _ln(t, g2_ref[0, 0], b2_ref[0, 0])
    t0_ref[0, 0:1] = cls_ref[0] + pos_ref[0, 0:1]
    t0_ref[0, 1:SEQ] = t + pos_ref[0, 1:SEQ]


def _embed(xp, g1, b1, pw, pb, g2, b2, cls, pos, se, si):
    espec = lambda *blk: pl.BlockSpec((1,) + blk, lambda i, se, si: (se[i],) + (0,) * len(blk))
    return pl.pallas_call(
        _embed_kernel,
        grid_spec=pltpu.PrefetchScalarGridSpec(
            num_scalar_prefetch=2,
            grid=(BATCH,),
            in_specs=[
                pl.BlockSpec((1, NPATCH, PATCH_DIM), lambda i, se, si: (si[i], 0, 0)),
                espec(1, PATCH_DIM), espec(1, PATCH_DIM),
                espec(PATCH_DIM, DIM), espec(1, DIM),
                espec(1, DIM), espec(1, DIM),
                espec(1, DIM),
                espec(SEQ, DIM),
            ],
            out_specs=pl.BlockSpec((1, SEQ, DIM), lambda i, se, si: (i, 0, 0)),
        ),
        out_shape=jax.ShapeDtypeStruct((BATCH, SEQ, DIM), jnp.float32),
    )(se, si, xp, g1, b1, pw, pb, g2, b2, cls, pos)


def _layers_kernel(se_ref, si_ref, t0_ref, alg_ref, alb_ref, qkv_ref,
                   ow_ref, ob_ref, flg_ref, flb_ref, w1_ref, b1_ref,
                   w2_ref, b2_ref, fg_ref, fb_ref, hw_ref, hb_ref,
                   out_ref, tbuf_ref):
    l = pl.program_id(0)
    i = pl.program_id(1)

    @pl.when(l == 0)
    def _():
        tbuf_ref[i] = t0_ref[0]

    t = tbuf_ref[i]
    y = _ln(t, alg_ref[0, 0, 0], alb_ref[0, 0, 0])
    qkv = _mm(y, qkv_ref[0, 0])
    scale = DIM_HEAD ** -0.5
    ohs = []
    for h in range(HEADS):
        qh = qkv[:, h * DIM_HEAD:(h + 1) * DIM_HEAD]
        kh = qkv[:, INNER + h * DIM_HEAD:INNER + (h + 1) * DIM_HEAD]
        vh = qkv[:, 2 * INNER + h * DIM_HEAD:2 * INNER + (h + 1) * DIM_HEAD]
        s = jax.lax.dot_general(
            qh, kh, (((1,), (1,)), ((), ())),
            precision=PREC, preferred_element_type=jnp.float32) * scale
        s = jax.nn.softmax(s, axis=-1)
        ohs.append(_mm(s, vh))
    o = jnp.concatenate(ohs, axis=1)
    t = t + _mm(o, ow_ref[0, 0]) + ob_ref[0, 0, 0]
    y = _ln(t, flg_ref[0, 0, 0], flb_ref[0, 0, 0])
    y = _mm(y, w1_ref[0, 0]) + b1_ref[0, 0, 0]
    y = 0.5 * y * (1.0 + jax.lax.erf(y * (2.0 ** -0.5)))
    t = t + _mm(y, w2_ref[0, 0]) + b2_ref[0, 0, 0]
    tbuf_ref[i] = t

    @pl.when(l == DEPTH - 1)
    def _():
        e = se_ref[i]
        tf = _ln(t[0:1, :], fg_ref[e], fb_ref[e])
        out_ref[0] = _mm(tf, hw_ref[e]) + hb_ref[pl.ds(e, 1)]


def _layers(t0, alg, alb, qkvw, ow, ob, flg, flb, w1, b1, w2, b2,
            fg, fb, hw, hb, se, si):
    lspec = lambda *blk: pl.BlockSpec(
        (1, 1) + blk, lambda l, i, se, si: (se[i], l) + (0,) * len(blk))
    vspec = lambda d: pl.BlockSpec(
        (1, 1, 1, d), lambda l, i, se, si: (se[i], l, 0, 0))
    full = lambda arr: pl.BlockSpec(arr.shape, lambda l, i, se, si: (0,) * arr.ndim)
    return pl.pallas_call(
        _layers_kernel,
        grid_spec=pltpu.PrefetchScalarGridSpec(
            num_scalar_prefetch=2,
            grid=(DEPTH, BATCH),
            in_specs=[
                pl.BlockSpec((1, SEQ, DIM),
                             lambda l, i, se, si: (jnp.where(l == 0, i, BATCH - 1), 0, 0)),
                vspec(DIM), vspec(DIM),
                lspec(DIM, 3 * INNER),
                lspec(INNER, DIM), vspec(DIM),
                vspec(DIM), vspec(DIM),
                lspec(DIM, MLP_DIM), vspec(MLP_DIM),
                lspec(MLP_DIM, DIM), vspec(DIM),
                full(fg), full(fb), full(hw), full(hb),
            ],
            out_specs=pl.BlockSpec(
                (1, 1, NUM_CLASSES),
                lambda l, i, se, si: (jnp.where(l == DEPTH - 1, si[i], si[0]), 0, 0)),
            scratch_shapes=[pltpu.VMEM((BATCH, SEQ, DIM), jnp.float32)],
        ),
        out_shape=jax.ShapeDtypeStruct((BATCH, 1, NUM_CLASSES), jnp.float32),
    )(se, si, t0, alg, alb, qkvw, ow, ob, flg, flb, w1, b1, w2, b2,
      fg, fb, hw, hb)


def kernel(x, params):
    experts = params['experts']
    stack = lambda key: jnp.stack([e[key] for e in experts])

    xf = x.reshape(BATCH, -1)
    h = SIZE // PATCH
    xp = x.reshape(BATCH, 3, h, PATCH, h, PATCH).transpose(
        0, 2, 4, 3, 5, 1).reshape(BATCH, NPATCH, PATCH_DIM)

    top1 = _gate(xf, params['gate_w'].T, params['gate_b'].reshape(1, -1))[:, 0]
    order = jnp.argsort(top1)
    se = top1[order].astype(jnp.int32)
    si = order.astype(jnp.int32)

    vec = lambda key: stack(key)[:, None, :]
    lvec = lambda key: stack(key)[:, :, None, :]

    t0 = _embed(
        xp,
        vec('pe_ln1_g'), vec('pe_ln1_b'), stack('pe_w'), vec('pe_b'),
        vec('pe_ln2_g'), vec('pe_ln2_b'),
        stack('cls').reshape(NUM_EXPERTS, 1, DIM),
        stack('pos').reshape(NUM_EXPERTS, SEQ, DIM),
        se, si)

    out = _layers(
        t0,
        lvec('attn_ln_g'), lvec('attn_ln_b'), stack('qkv_w'),
        stack('out_w'), lvec('out_b'),
        lvec('ff_ln_g'), lvec('ff_ln_b'),
        stack('w1'), lvec('b1'), stack('w2'), lvec('b2'),
        stack('final_ln_g'), stack('final_ln_b'),
        stack('head_w'), stack('head_b'),
        se, si)
    return out.reshape(BATCH, NUM_CLASSES)


# speedup vs baseline: 6.0640x; 1.3032x over previous
import functools

import jax
import jax.numpy as jnp
from jax.experimental import pallas as pl
from jax.experimental.pallas import tpu as pltpu

NUM_EXPERTS = 8
SIZE = 224
PATCH = 16
DIM = 384
DEPTH = 6
HEADS = 8
DIM_HEAD = 64
MLP_DIM = 512
NUM_CLASSES = 10
BATCH = 32
NPATCH = (SIZE // PATCH) ** 2
PATCH_DIM = 3 * PATCH * PATCH
INNER = HEADS * DIM_HEAD
SEQ = NPATCH + 1

GATE_PREC = jax.lax.Precision.HIGHEST

GATE_CHUNKS = 8
GATE_K = 3 * SIZE * SIZE // GATE_CHUNKS


def _split_bf16(a):
    hi = a.astype(jnp.bfloat16)
    lo = (a - hi.astype(jnp.float32)).astype(jnp.bfloat16)
    return hi, lo


def _dot3(a, b, dims):
    ah, al = _split_bf16(a)
    bh, bl = _split_bf16(b)
    d = lambda u, v: jax.lax.dot_general(
        u, v, dims, preferred_element_type=jnp.float32)
    return d(ah, bh) + d(ah, bl) + d(al, bh)


def _mm(a, b):
    return _dot3(a, b, (((a.ndim - 1,), (0,)), ((), ())))


def _ln(x, g, b):
    m = jnp.mean(x, axis=-1, keepdims=True)
    v = jnp.mean((x - m) ** 2, axis=-1, keepdims=True)
    return (x - m) * jax.lax.rsqrt(v + 1e-5) * g + b


def _gate_kernel(xf_ref, gwt_ref, gb_ref, top1_ref, acc_ref):
    k = pl.program_id(0)

    @pl.when(k == 0)
    def _():
        acc_ref[...] = jnp.broadcast_to(gb_ref[...], (BATCH, NUM_EXPERTS))

    acc_ref[...] += jax.lax.dot_general(
        xf_ref[...], gwt_ref[...], (((1,), (1,)), ((), ())),
        precision=GATE_PREC, preferred_element_type=jnp.float32)

    @pl.when(k == GATE_CHUNKS - 1)
    def _():
        logits = acc_ref[...]
        m = jnp.max(logits, axis=1, keepdims=True)
        col = jax.lax.broadcasted_iota(jnp.int32, (BATCH, NUM_EXPERTS), 1)
        top1_ref[...] = jnp.min(
            jnp.where(logits == m, col, NUM_EXPERTS), axis=1, keepdims=True)


def _gate(xf, gwt, gb):
    return pl.pallas_call(
        _gate_kernel,
        grid=(GATE_CHUNKS,),
        in_specs=[
            pl.BlockSpec((BATCH, GATE_K), lambda k: (0, k)),
            pl.BlockSpec((NUM_EXPERTS, GATE_K), lambda k: (0, k)),
            pl.BlockSpec((1, NUM_EXPERTS), lambda k: (0, 0)),
        ],
        out_specs=pl.BlockSpec((BATCH, 1), lambda k: (0, 0)),
        out_shape=jax.ShapeDtypeStruct((BATCH, 1), jnp.int32),
        scratch_shapes=[pltpu.VMEM((BATCH, NUM_EXPERTS), jnp.float32)],
    )(xf, gwt, gb)


def _embed_kernel(se_ref, si_ref, xp_ref, g1_ref, b1_ref, pw_ref, pb_ref,
                  g2_ref, b2_ref, cls_ref, pos_ref, t0_ref):
    p = _ln(xp_ref[0], g1_ref[0, 0], b1_ref[0, 0])
    t = _mm(p, pw_ref[0]) + pb_ref[0, 0]
    t = _ln(t, g2_ref[0, 0], b2_ref[0, 0])
    t0_ref[0, 0:1] = cls_ref[0] + pos_ref[0, 0:1]
    t0_ref[0, 1:SEQ] = t + pos_ref[0, 1:SEQ]


def _embed(xp, g1, b1, pw, pb, g2, b2, cls, pos, se, si):
    espec = lambda *blk: pl.BlockSpec((1,) + blk, lambda i, se, si: (se[i],) + (0,) * len(blk))
    return pl.pallas_call(
        _embed_kernel,
        grid_spec=pltpu.PrefetchScalarGridSpec(
            num_scalar_prefetch=2,
            grid=(BATCH,),
            in_specs=[
                pl.BlockSpec((1, NPATCH, PATCH_DIM), lambda i, se, si: (si[i], 0, 0)),
                espec(1, PATCH_DIM), espec(1, PATCH_DIM),
                espec(PATCH_DIM, DIM), espec(1, DIM),
                espec(1, DIM), espec(1, DIM),
                espec(1, DIM),
                espec(SEQ, DIM),
            ],
            out_specs=pl.BlockSpec((1, SEQ, DIM), lambda i, se, si: (i, 0, 0)),
        ),
        out_shape=jax.ShapeDtypeStruct((BATCH, SEQ, DIM), jnp.float32),
    )(se, si, xp, g1, b1, pw, pb, g2, b2, cls, pos)


def _layers_kernel(se_ref, si_ref, t0_ref, alg_ref, alb_ref, qkv_ref,
                   ow_ref, ob_ref, flg_ref, flb_ref, w1_ref, b1_ref,
                   w2_ref, b2_ref, fg_ref, fb_ref, hw_ref, hb_ref,
                   out_ref, tbuf_ref):
    l = pl.program_id(0)
    i = pl.program_id(1)

    @pl.when(l == 0)
    def _():
        tbuf_ref[i] = t0_ref[0]

    t = tbuf_ref[i]
    y = _ln(t, alg_ref[0, 0, 0], alb_ref[0, 0, 0])
    qkv = _mm(y, qkv_ref[0, 0])
    scale = DIM_HEAD ** -0.5
    ohs = []
    for h in range(HEADS):
        qh = qkv[:, h * DIM_HEAD:(h + 1) * DIM_HEAD]
        kh = qkv[:, INNER + h * DIM_HEAD:INNER + (h + 1) * DIM_HEAD]
        vh = qkv[:, 2 * INNER + h * DIM_HEAD:2 * INNER + (h + 1) * DIM_HEAD]
        s = _dot3(qh, kh, (((1,), (1,)), ((), ()))) * scale
        s = jax.nn.softmax(s, axis=-1)
        ohs.append(_mm(s, vh))
    o = jnp.concatenate(ohs, axis=1)
    t = t + _mm(o, ow_ref[0, 0]) + ob_ref[0, 0, 0]
    y = _ln(t, flg_ref[0, 0, 0], flb_ref[0, 0, 0])
    y = _mm(y, w1_ref[0, 0]) + b1_ref[0, 0, 0]
    y = 0.5 * y * (1.0 + jax.lax.erf(y * (2.0 ** -0.5)))
    t = t + _mm(y, w2_ref[0, 0]) + b2_ref[0, 0, 0]
    tbuf_ref[i] = t

    @pl.when(l == DEPTH - 1)
    def _():
        e = se_ref[i]
        tf = _ln(t[0:1, :], fg_ref[e], fb_ref[e])
        out_ref[0] = _mm(tf, hw_ref[e]) + hb_ref[pl.ds(e, 1)]


def _layers(t0, alg, alb, qkvw, ow, ob, flg, flb, w1, b1, w2, b2,
            fg, fb, hw, hb, se, si):
    lspec = lambda *blk: pl.BlockSpec(
        (1, 1) + blk, lambda l, i, se, si: (se[i], l) + (0,) * len(blk))
    vspec = lambda d: pl.BlockSpec(
        (1, 1, 1, d), lambda l, i, se, si: (se[i], l, 0, 0))
    full = lambda arr: pl.BlockSpec(arr.shape, lambda l, i, se, si: (0,) * arr.ndim)
    return pl.pallas_call(
        _layers_kernel,
        grid_spec=pltpu.PrefetchScalarGridSpec(
            num_scalar_prefetch=2,
            grid=(DEPTH, BATCH),
            in_specs=[
                pl.BlockSpec((1, SEQ, DIM),
                             lambda l, i, se, si: (jnp.where(l == 0, i, BATCH - 1), 0, 0)),
                vspec(DIM), vspec(DIM),
                lspec(DIM, 3 * INNER),
                lspec(INNER, DIM), vspec(DIM),
                vspec(DIM), vspec(DIM),
                lspec(DIM, MLP_DIM), vspec(MLP_DIM),
                lspec(MLP_DIM, DIM), vspec(DIM),
                full(fg), full(fb), full(hw), full(hb),
            ],
            out_specs=pl.BlockSpec(
                (1, 1, NUM_CLASSES),
                lambda l, i, se, si: (jnp.where(l == DEPTH - 1, si[i], si[0]), 0, 0)),
            scratch_shapes=[pltpu.VMEM((BATCH, SEQ, DIM), jnp.float32)],
        ),
        out_shape=jax.ShapeDtypeStruct((BATCH, 1, NUM_CLASSES), jnp.float32),
    )(se, si, t0, alg, alb, qkvw, ow, ob, flg, flb, w1, b1, w2, b2,
      fg, fb, hw, hb)


def kernel(x, params):
    experts = params['experts']
    stack = lambda key: jnp.stack([e[key] for e in experts])

    xf = x.reshape(BATCH, -1)
    h = SIZE // PATCH
    xp = x.reshape(BATCH, 3, h, PATCH, h, PATCH).transpose(
        0, 2, 4, 3, 5, 1).reshape(BATCH, NPATCH, PATCH_DIM)

    top1 = _gate(xf, params['gate_w'].T, params['gate_b'].reshape(1, -1))[:, 0]
    order = jnp.argsort(top1)
    se = top1[order].astype(jnp.int32)
    si = order.astype(jnp.int32)

    vec = lambda key: stack(key)[:, None, :]
    lvec = lambda key: stack(key)[:, :, None, :]

    t0 = _embed(
        xp,
        vec('pe_ln1_g'), vec('pe_ln1_b'), stack('pe_w'), vec('pe_b'),
        vec('pe_ln2_g'), vec('pe_ln2_b'),
        stack('cls').reshape(NUM_EXPERTS, 1, DIM),
        stack('pos').reshape(NUM_EXPERTS, SEQ, DIM),
        se, si)

    out = _layers(
        t0,
        lvec('attn_ln_g'), lvec('attn_ln_b'), stack('qkv_w'),
        stack('out_w'), lvec('out_b'),
        lvec('ff_ln_g'), lvec('ff_ln_b'),
        stack('w1'), lvec('b1'), stack('w2'), lvec('b2'),
        stack('final_ln_g'), stack('final_ln_b'),
        stack('head_w'), stack('head_b'),
        se, si)
    return out.reshape(BATCH, NUM_CLASSES)
